# baseline (speedup 1.0000x reference)
import functools

import jax
import jax.numpy as jnp
from jax import lax
from jax.experimental import pallas as pl
from jax.experimental.pallas import tpu as pltpu
from jax.experimental.pallas import tpu_sc as plsc

NUM_WORKERS = 32
R_ITEMS = 4
NBUF = 2
SPLITS = ((0, 96), (96, 104))


def _make_gather(batch, hist, d_model):
    per_w = batch // NUM_WORKERS
    n_it = per_w // (R_ITEMS * NBUF)
    assert per_w % (R_ITEMS * NBUF) == 0 and hist == 200
    mesh = plsc.VectorSubcoreMesh(core_axis_name="c", subcore_axis_name="s")

    @functools.partial(
        pl.kernel,
        out_type=jax.ShapeDtypeStruct((batch, hist, d_model), jnp.float32),
        mesh=mesh,
        scratch_types=[
            pltpu.VMEM((NBUF, R_ITEMS, hist), jnp.int32),
            pltpu.VMEM((NBUF, R_ITEMS, hist, d_model), jnp.float32),
            pltpu.SemaphoreType.DMA,
            pltpu.SemaphoreType.DMA,
            pltpu.SemaphoreType.DMA,
        ],
        compiler_params=pltpu.CompilerParams(use_tc_tiling_on_sc=False),
    )
    def k(table_hbm, idx_hbm, out_hbm, idx_v, rows_v, gsem, osem0, osem1):
        osems = (osem0, osem1)
        wid = lax.axis_index("s") * 2 + lax.axis_index("c")
        base = wid * per_w

        def outer(t, carry):
            for b in range(NBUF):
                i0 = base + (t * NBUF + b) * R_ITEMS

                @pl.when(t > 0)
                def _wait_prev_scatter():
                    pltpu.make_async_copy(
                        rows_v.at[b], out_hbm.at[pl.ds(0, R_ITEMS)], osems[b]
                    ).wait()

                pltpu.sync_copy(idx_hbm.at[pl.ds(i0, R_ITEMS)], idx_v.at[b])
                descs = [
                    pltpu.async_copy(
                        table_hbm.at[idx_v.at[b].at[r].at[pl.ds(off, ln)]],
                        rows_v.at[b].at[r].at[pl.ds(off, ln)],
                        gsem,
                    )
                    for r in range(R_ITEMS)
                    for (off, ln) in SPLITS
                ]
                for d in descs:
                    d.wait()
                pltpu.async_copy(
                    rows_v.at[b], out_hbm.at[pl.ds(i0, R_ITEMS)], osems[b]
                )
            return carry

        lax.fori_loop(0, n_it, outer, 0)
        for b in range(NBUF):
            pltpu.make_async_copy(
                rows_v.at[b], out_hbm.at[pl.ds(0, R_ITEMS)], osems[b]
            ).wait()

    return k


def kernel(cumulative_positions, position_embeddings):
    b, h = cumulative_positions.shape
    d = position_embeddings.shape[1]
    idx = cumulative_positions.astype(jnp.int32)
    return _make_gather(b, h, d)(position_embeddings, idx)
